# Initial kernel scaffold; baseline (speedup 1.0000x reference)
import jax
import jax.numpy as jnp
from jax.experimental import pallas as pl


def kernel(x, Wg, Wnoise, W1, b1, W2, b2):
    raise NotImplementedError("write your pallas kernel here")


# speedup vs baseline: 1.0073x; 1.0073x over previous
import jax
import jax.numpy as jnp
from jax.experimental import pallas as pl
from jax.experimental.pallas import tpu as pltpu

D_IN = 768
D_HID = 3072
N_EXP = 16


def _moe_kernel(x_ref, Wg_ref, Wn_ref, eps_ref, b1_ref, b2_ref,
                W1_ref, W2_ref, out_ref, w_scr):
    e = pl.program_id(0)
    xv = x_ref[...]

    @pl.when(e == 0)
    def _init():
        gl = jnp.dot(xv, Wg_ref[...].T, preferred_element_type=jnp.float32)
        nl = jnp.dot(xv, Wn_ref[...].T, preferred_element_type=jnp.float32)
        logits = gl + jax.nn.softplus(nl) * eps_ref[...]
        eidx = jax.lax.broadcasted_iota(jnp.int32, logits.shape, 1)
        v1 = jnp.max(logits, axis=-1, keepdims=True)
        i1 = jnp.argmax(logits, axis=-1)[:, None]
        masked = jnp.where(eidx == i1, -jnp.inf, logits)
        i2 = jnp.argmax(masked, axis=-1)[:, None]
        sel = (eidx == i1) | (eidx == i2)
        ew = jnp.where(sel, jnp.exp(logits - v1), 0.0)
        w = ew / jnp.sum(ew, axis=-1, keepdims=True)
        w_scr[...] = w
        out_ref[...] = jnp.dot(w, b2_ref[...], preferred_element_type=jnp.float32)

    eidx = jax.lax.broadcasted_iota(jnp.int32, (32, N_EXP), 1)
    w_col = jnp.sum(jnp.where(eidx == e, w_scr[...], 0.0), axis=1, keepdims=True)
    h = jnp.dot(xv, W1_ref[0], preferred_element_type=jnp.float32)
    h = jnp.maximum(h + b1_ref[pl.ds(e, 1), :], 0.0)
    out_ref[...] += jnp.dot(w_col * h, W2_ref[0],
                            preferred_element_type=jnp.float32)


def kernel(x, Wg, Wnoise, W1, b1, W2, b2):
    b, c, d = x.shape
    xm = x.reshape(b * c, d)
    eps = jax.random.normal(jax.random.key(42), (b * c, N_EXP), dtype=x.dtype)

    out = pl.pallas_call(
        _moe_kernel,
        grid=(N_EXP,),
        in_specs=[
            pl.BlockSpec((b * c, D_IN), lambda e: (0, 0)),
            pl.BlockSpec((N_EXP, D_IN), lambda e: (0, 0)),
            pl.BlockSpec((N_EXP, D_IN), lambda e: (0, 0)),
            pl.BlockSpec((b * c, N_EXP), lambda e: (0, 0)),
            pl.BlockSpec((N_EXP, D_HID), lambda e: (0, 0)),
            pl.BlockSpec((N_EXP, D_IN), lambda e: (0, 0)),
            pl.BlockSpec((1, D_IN, D_HID), lambda e: (e, 0, 0)),
            pl.BlockSpec((1, D_HID, D_IN), lambda e: (e, 0, 0)),
        ],
        out_specs=pl.BlockSpec((b * c, D_IN), lambda e: (0, 0)),
        out_shape=jax.ShapeDtypeStruct((b * c, D_IN), jnp.float32),
        scratch_shapes=[pltpu.VMEM((b * c, N_EXP), jnp.float32)],
    )(xm, Wg, Wnoise, eps, b1, b2, W1, W2)
    return out.reshape(b, c, d)


# speedup vs baseline: 1.0472x; 1.0395x over previous
import jax
import jax.numpy as jnp
from jax.experimental import pallas as pl
from jax.experimental.pallas import tpu as pltpu

D_IN = 768
D_HID = 3072
N_EXP = 16
N_HC = 2
H_BLK = D_HID // N_HC


def _moe_kernel(x_ref, Wg_ref, Wn_ref, eps_ref, b1_ref, b2_ref,
                W1_ref, W2_ref, out_ref, w_scr):
    e = pl.program_id(0)
    hc = pl.program_id(1)
    xv = x_ref[...]

    @pl.when((e == 0) & (hc == 0))
    def _init():
        gl = jnp.dot(xv, Wg_ref[...].T, preferred_element_type=jnp.float32)
        nl = jnp.dot(xv, Wn_ref[...].T, preferred_element_type=jnp.float32)
        logits = gl + jax.nn.softplus(nl) * eps_ref[...]
        eidx = jax.lax.broadcasted_iota(jnp.int32, logits.shape, 1)
        v1 = jnp.max(logits, axis=-1, keepdims=True)
        i1 = jnp.argmax(logits, axis=-1)[:, None]
        masked = jnp.where(eidx == i1, -jnp.inf, logits)
        i2 = jnp.argmax(masked, axis=-1)[:, None]
        sel = (eidx == i1) | (eidx == i2)
        ew = jnp.where(sel, jnp.exp(logits - v1), 0.0)
        w = ew / jnp.sum(ew, axis=-1, keepdims=True)
        w_scr[...] = w
        out_ref[...] = jnp.dot(w, b2_ref[...], preferred_element_type=jnp.float32)

    eidx = jax.lax.broadcasted_iota(jnp.int32, (32, N_EXP), 1)
    w_col = jnp.sum(jnp.where(eidx == e, w_scr[...], 0.0), axis=1, keepdims=True)
    h = jnp.dot(xv, W1_ref[0], preferred_element_type=jnp.float32)
    h = jnp.maximum(h + b1_ref[pl.ds(e, 1), pl.ds(hc * H_BLK, H_BLK)], 0.0)
    out_ref[...] += jnp.dot(w_col * h, W2_ref[0],
                            preferred_element_type=jnp.float32)


def kernel(x, Wg, Wnoise, W1, b1, W2, b2):
    b, c, d = x.shape
    xm = x.reshape(b * c, d)
    eps = jax.random.normal(jax.random.key(42), (b * c, N_EXP), dtype=x.dtype)

    out = pl.pallas_call(
        _moe_kernel,
        grid=(N_EXP, N_HC),
        in_specs=[
            pl.BlockSpec((b * c, D_IN), lambda e, hc: (0, 0)),
            pl.BlockSpec((N_EXP, D_IN), lambda e, hc: (0, 0)),
            pl.BlockSpec((N_EXP, D_IN), lambda e, hc: (0, 0)),
            pl.BlockSpec((b * c, N_EXP), lambda e, hc: (0, 0)),
            pl.BlockSpec((N_EXP, D_HID), lambda e, hc: (0, 0)),
            pl.BlockSpec((N_EXP, D_IN), lambda e, hc: (0, 0)),
            pl.BlockSpec((1, D_IN, H_BLK), lambda e, hc: (e, 0, hc)),
            pl.BlockSpec((1, H_BLK, D_IN), lambda e, hc: (e, hc, 0)),
        ],
        out_specs=pl.BlockSpec((b * c, D_IN), lambda e, hc: (0, 0)),
        out_shape=jax.ShapeDtypeStruct((b * c, D_IN), jnp.float32),
        scratch_shapes=[pltpu.VMEM((b * c, N_EXP), jnp.float32)],
    )(xm, Wg, Wnoise, eps, b1, b2, W1, W2)
    return out.reshape(b, c, d)
